# baseline (speedup 1.0000x reference)
import functools

import jax
import jax.numpy as jnp
from jax import lax
from jax.experimental import pallas as pl
from jax.experimental.pallas import tpu as pltpu
from jax.experimental.pallas import tpu_sc as plsc

N = 10000
E = 320000
D = 128

NC = 2
NS = 16
NW = NC * NS
EPW = E // NW
CH = 80
NCH = EPW // CH
NPAIR = (NCH - 1) // 2
RPT = 632
NP = NS * RPT

_mesh = plsc.VectorSubcoreMesh(core_axis_name="c", subcore_axis_name="s")


def _aggregate_body(with_count, x_hbm, src_hbm, dst_hbm, zeros_hbm,
                    zeros_np_hbm, out_hbm, cnt_hbm, srcv, idx_d0, idx_d1,
                    rows0, rows1, ones, cnt_stage, acc, cnt_acc,
                    g0, g1, d0, d1, s0, s1, c0, c1):
    cid = lax.axis_index("c")
    sid = lax.axis_index("s")
    wid = sid * NC + cid
    ebase = wid * EPW

    pltpu.sync_copy(src_hbm.at[pl.ds(ebase, EPW)], srcv)
    pltpu.sync_copy(zeros_hbm, acc.at[pl.ds(sid * RPT, RPT)])
    if with_count:
        for j in range(CH // 16):
            ones[pl.ds(j * 16, 16)] = jnp.ones((16,), jnp.float32)

        @pl.when(sid == 0)
        def _():
            pltpu.sync_copy(zeros_np_hbm, cnt_stage)
            pltpu.sync_copy(cnt_stage, cnt_acc)
    plsc.subcore_barrier()

    def fetch_d(j, buf, sem):
        pltpu.async_copy(dst_hbm.at[pl.ds(ebase + j * CH, CH)], buf, sem)

    def wait_d(j, buf, sem):
        pltpu.make_async_copy(dst_hbm.at[pl.ds(ebase + j * CH, CH)],
                              buf, sem).wait()

    def issue_g(j, buf, sem):
        pltpu.async_copy(x_hbm.at[srcv.at[pl.ds(j * CH, CH)]], buf, sem)

    def wait_g(j, buf, sem):
        pltpu.make_async_copy(x_hbm.at[srcv.at[pl.ds(j * CH, CH)]],
                              buf, sem).wait()

    def scat(idx_buf, buf, sem):
        pltpu.async_copy(buf, acc.at[idx_buf], sem, add=True)

    def wait_scat(idx_buf, buf, sem):
        pltpu.make_async_copy(buf, acc.at[idx_buf], sem).wait()

    def scat_cnt(idx_buf, sem):
        if with_count:
            pltpu.async_copy(ones, cnt_acc.at[idx_buf], sem, add=True)

    def wait_scat_cnt(idx_buf, sem):
        if with_count:
            pltpu.make_async_copy(ones, cnt_acc.at[idx_buf], sem).wait()

    fetch_d(0, idx_d0, d0)
    issue_g(0, rows0, g0)
    fetch_d(1, idx_d1, d1)
    issue_g(1, rows1, g1)

    def body(k, carry):
        j0 = 2 * k
        j1 = j0 + 1
        wait_g(j0, rows0, g0)
        wait_d(j0, idx_d0, d0)
        scat(idx_d0, rows0, s0)
        scat_cnt(idx_d0, c0)
        wait_g(j1, rows1, g1)
        wait_d(j1, idx_d1, d1)
        scat(idx_d1, rows1, s1)
        scat_cnt(idx_d1, c1)
        wait_scat(idx_d0, rows0, s0)
        wait_scat_cnt(idx_d0, c0)
        fetch_d(j0 + 2, idx_d0, d0)
        issue_g(j0 + 2, rows0, g0)
        wait_scat(idx_d1, rows1, s1)
        wait_scat_cnt(idx_d1, c1)

        @pl.when(k < NPAIR - 1)
        def _():
            fetch_d(j0 + 3, idx_d1, d1)
            issue_g(j0 + 3, rows1, g1)

        return carry

    lax.fori_loop(0, NPAIR, body, 0)
    wait_g(NCH - 1, rows0, g0)
    wait_d(NCH - 1, idx_d0, d0)
    scat(idx_d0, rows0, s0)
    scat_cnt(idx_d0, c0)
    wait_scat(idx_d0, rows0, s0)
    wait_scat_cnt(idx_d0, c0)
    plsc.subcore_barrier()

    pltpu.sync_copy(acc.at[pl.ds(sid * RPT, RPT)],
                    out_hbm.at[pl.ds(cid * NP + sid * RPT, RPT)])
    if with_count:
        @pl.when(sid == 0)
        def _():
            pltpu.sync_copy(cnt_acc, cnt_stage)
            pltpu.sync_copy(cnt_stage, cnt_hbm.at[pl.ds(cid * NP, NP)])


_AGG_SCRATCH = [
    pltpu.VMEM((EPW,), jnp.int32),
    pltpu.VMEM((CH,), jnp.int32),
    pltpu.VMEM((CH,), jnp.int32),
    pltpu.VMEM((CH, D), jnp.float32),
    pltpu.VMEM((CH, D), jnp.float32),
    pltpu.VMEM((CH,), jnp.float32),
    pltpu.VMEM((NP,), jnp.float32),
    pltpu.VMEM_SHARED((NP, D), jnp.float32),
    pltpu.VMEM_SHARED((NP,), jnp.float32),
    pltpu.SemaphoreType.DMA,
    pltpu.SemaphoreType.DMA,
    pltpu.SemaphoreType.DMA,
    pltpu.SemaphoreType.DMA,
    pltpu.SemaphoreType.DMA,
    pltpu.SemaphoreType.DMA,
    pltpu.SemaphoreType.DMA,
    pltpu.SemaphoreType.DMA,
]

_sc_aggregate_count = functools.partial(
    pl.kernel,
    out_type=(jax.ShapeDtypeStruct((2 * NP, D), jnp.float32),
              jax.ShapeDtypeStruct((2 * NP,), jnp.float32)),
    mesh=_mesh,
    scratch_types=_AGG_SCRATCH,
)(lambda x, s, d, z, zn, o, c, *r: _aggregate_body(True, x, s, d, z, zn,
                                                   o, c, *r))


def _agg_only_body(x, s, d, z, o, *r):
    return _aggregate_body(False, x, s, d, z, None, o, None, *r)


_sc_aggregate = functools.partial(
    pl.kernel,
    out_type=jax.ShapeDtypeStruct((2 * NP, D), jnp.float32),
    mesh=_mesh,
    scratch_types=_AGG_SCRATCH,
)(_agg_only_body)


BN = 1000


def _combine_body(act, p_ref, cnt_ref, h_ref, wl_ref, bl_ref, wr_ref, o_ref):
    s = p_ref[0] + p_ref[1]
    c = cnt_ref[0] + cnt_ref[1]
    mean = s / jnp.maximum(c, 1.0)
    a = lax.dot_general(mean, wl_ref[...], (((1,), (1,)), ((), ())),
                        preferred_element_type=jnp.float32)
    b = lax.dot_general(h_ref[...], wr_ref[...], (((1,), (1,)), ((), ())),
                        preferred_element_type=jnp.float32)
    o = a + b + bl_ref[...]
    if act == "relu":
        o = jnp.maximum(o, 0.0)
    else:
        o = jnp.tanh(o)
    o_ref[...] = o


def _make_combine(act):
    return pl.pallas_call(
        functools.partial(_combine_body, act),
        grid=(N // BN,),
        in_specs=[
            pl.BlockSpec((2, BN, D), lambda i: (0, i, 0)),
            pl.BlockSpec((2, BN, 1), lambda i: (0, i, 0)),
            pl.BlockSpec((BN, D), lambda i: (i, 0)),
            pl.BlockSpec((D, D), lambda i: (0, 0)),
            pl.BlockSpec((1, D), lambda i: (0, 0)),
            pl.BlockSpec((D, D), lambda i: (0, 0)),
        ],
        out_specs=pl.BlockSpec((BN, D), lambda i: (i, 0)),
        out_shape=jax.ShapeDtypeStruct((N, D), jnp.float32),
    )


_combine_relu = _make_combine("relu")
_combine_tanh = _make_combine("tanh")


def kernel(x, edge_index, g, Wl0, bl0, Wr0, Wl1, bl1, Wr1, Wl2, bl2, Wr2):
    src = edge_index[0]
    dst = edge_index[1]
    zeros_rows = jnp.zeros((RPT, D), jnp.float32)
    zeros_np = jnp.zeros((NP,), jnp.float32)

    p0, cnt2 = _sc_aggregate_count(x, src, dst, zeros_rows, zeros_np)
    cnt = cnt2.reshape(2, NP, 1)

    h = x
    layers = [(Wl0, bl0, Wr0, _combine_relu, None),
              (Wl1, bl1, Wr1, _combine_relu, None),
              (Wl2, bl2, Wr2, _combine_tanh, None)]
    for i, (Wl, bl, Wr, combine, _) in enumerate(layers):
        if i == 0:
            p = p0.reshape(2, NP, D)
        else:
            p = _sc_aggregate(h, src, dst, zeros_rows).reshape(2, NP, D)
        h = combine(p, cnt, h, Wl, bl.reshape(1, D), Wr)
    return h


# speedup vs baseline: 1.2034x; 1.2034x over previous
import functools

import jax
import jax.numpy as jnp
from jax import lax
from jax.experimental import pallas as pl
from jax.experimental.pallas import tpu as pltpu
from jax.experimental.pallas import tpu_sc as plsc

N = 10000
E = 320000
D = 128

NC = 2
NS = 16
NW = NC * NS
EPW = E // NW
CH = 80
NCH = EPW // CH
NPAIR = (NCH - 1) // 2
RPT = 632
NP = NS * RPT

_mesh = plsc.VectorSubcoreMesh(core_axis_name="c", subcore_axis_name="s")


def _aggregate_body(with_count, x_hbm, src_hbm, dst_hbm, zeros_hbm,
                    zeros_np_hbm, out_hbm, cnt_hbm, srcv, idx_d0, idx_d1,
                    rows0, rows1, ones, cnt_stage, acc, cnt_acc,
                    g0, g1, d0, d1, s0, s1, c0, c1):
    cid = lax.axis_index("c")
    sid = lax.axis_index("s")
    wid = sid * NC + cid
    ebase = wid * EPW

    pltpu.sync_copy(src_hbm.at[pl.ds(ebase, EPW)], srcv)
    pltpu.sync_copy(zeros_hbm, acc.at[pl.ds(sid * RPT, RPT)])
    if with_count:
        for j in range(CH // 16):
            ones[pl.ds(j * 16, 16)] = jnp.ones((16,), jnp.float32)

        @pl.when(sid == 0)
        def _():
            pltpu.sync_copy(zeros_np_hbm, cnt_stage)
            pltpu.sync_copy(cnt_stage, cnt_acc)
    plsc.subcore_barrier()

    def fetch_d(j, buf, sem):
        pltpu.async_copy(dst_hbm.at[pl.ds(ebase + j * CH, CH)], buf, sem)

    def wait_d(j, buf, sem):
        pltpu.make_async_copy(dst_hbm.at[pl.ds(ebase + j * CH, CH)],
                              buf, sem).wait()

    def issue_g(j, buf, sem):
        pltpu.async_copy(x_hbm.at[srcv.at[pl.ds(j * CH, CH)]], buf, sem)

    def wait_g(j, buf, sem):
        pltpu.make_async_copy(x_hbm.at[srcv.at[pl.ds(j * CH, CH)]],
                              buf, sem).wait()

    def scat_sync(idx_buf, buf):
        pltpu.sync_copy(buf, acc.at[idx_buf], add=True)

    def scat_cnt(idx_buf, sem):
        if with_count:
            pltpu.async_copy(ones, cnt_acc.at[idx_buf], sem, add=True)

    def wait_scat_cnt(idx_buf, sem):
        if with_count:
            pltpu.make_async_copy(ones, cnt_acc.at[idx_buf], sem).wait()

    fetch_d(0, idx_d0, d0)
    issue_g(0, rows0, g0)

    def body(k, carry):
        j0 = 2 * k
        j1 = j0 + 1
        fetch_d(j1, idx_d1, d1)
        issue_g(j1, rows1, g1)
        wait_g(j0, rows0, g0)
        wait_d(j0, idx_d0, d0)
        scat_cnt(idx_d0, c0)
        scat_sync(idx_d0, rows0)
        wait_scat_cnt(idx_d0, c0)
        fetch_d(j0 + 2, idx_d0, d0)
        issue_g(j0 + 2, rows0, g0)
        wait_g(j1, rows1, g1)
        wait_d(j1, idx_d1, d1)
        scat_cnt(idx_d1, c1)
        scat_sync(idx_d1, rows1)
        wait_scat_cnt(idx_d1, c1)
        return carry

    lax.fori_loop(0, NPAIR, body, 0)
    wait_g(NCH - 1, rows0, g0)
    wait_d(NCH - 1, idx_d0, d0)
    scat_cnt(idx_d0, c0)
    scat_sync(idx_d0, rows0)
    wait_scat_cnt(idx_d0, c0)
    plsc.subcore_barrier()

    pltpu.sync_copy(acc.at[pl.ds(sid * RPT, RPT)],
                    out_hbm.at[pl.ds(cid * NP + sid * RPT, RPT)])
    if with_count:
        @pl.when(sid == 0)
        def _():
            pltpu.sync_copy(cnt_acc, cnt_stage)
            pltpu.sync_copy(cnt_stage, cnt_hbm.at[pl.ds(cid * NP, NP)])


_AGG_SCRATCH = [
    pltpu.VMEM((EPW,), jnp.int32),
    pltpu.VMEM((CH,), jnp.int32),
    pltpu.VMEM((CH,), jnp.int32),
    pltpu.VMEM((CH, D), jnp.float32),
    pltpu.VMEM((CH, D), jnp.float32),
    pltpu.VMEM((CH,), jnp.float32),
    pltpu.VMEM((NP,), jnp.float32),
    pltpu.VMEM_SHARED((NP, D), jnp.float32),
    pltpu.VMEM_SHARED((NP,), jnp.float32),
    pltpu.SemaphoreType.DMA,
    pltpu.SemaphoreType.DMA,
    pltpu.SemaphoreType.DMA,
    pltpu.SemaphoreType.DMA,
    pltpu.SemaphoreType.DMA,
    pltpu.SemaphoreType.DMA,
    pltpu.SemaphoreType.DMA,
    pltpu.SemaphoreType.DMA,
]

_sc_aggregate_count = functools.partial(
    pl.kernel,
    out_type=(jax.ShapeDtypeStruct((2 * NP, D), jnp.float32),
              jax.ShapeDtypeStruct((2 * NP,), jnp.float32)),
    mesh=_mesh,
    scratch_types=_AGG_SCRATCH,
)(lambda x, s, d, z, zn, o, c, *r: _aggregate_body(True, x, s, d, z, zn,
                                                   o, c, *r))


def _agg_only_body(x, s, d, z, o, *r):
    return _aggregate_body(False, x, s, d, z, None, o, None, *r)


_sc_aggregate = functools.partial(
    pl.kernel,
    out_type=jax.ShapeDtypeStruct((2 * NP, D), jnp.float32),
    mesh=_mesh,
    scratch_types=_AGG_SCRATCH,
)(_agg_only_body)


BN = 1000


def _combine_body(act, p_ref, cnt_ref, h_ref, wl_ref, bl_ref, wr_ref, o_ref):
    s = p_ref[0] + p_ref[1]
    c = cnt_ref[0] + cnt_ref[1]
    mean = s / jnp.maximum(c, 1.0)
    a = lax.dot_general(mean, wl_ref[...], (((1,), (1,)), ((), ())),
                        preferred_element_type=jnp.float32)
    b = lax.dot_general(h_ref[...], wr_ref[...], (((1,), (1,)), ((), ())),
                        preferred_element_type=jnp.float32)
    o = a + b + bl_ref[...]
    if act == "relu":
        o = jnp.maximum(o, 0.0)
    else:
        o = jnp.tanh(o)
    o_ref[...] = o


def _make_combine(act):
    return pl.pallas_call(
        functools.partial(_combine_body, act),
        grid=(N // BN,),
        in_specs=[
            pl.BlockSpec((2, BN, D), lambda i: (0, i, 0)),
            pl.BlockSpec((2, BN, 1), lambda i: (0, i, 0)),
            pl.BlockSpec((BN, D), lambda i: (i, 0)),
            pl.BlockSpec((D, D), lambda i: (0, 0)),
            pl.BlockSpec((1, D), lambda i: (0, 0)),
            pl.BlockSpec((D, D), lambda i: (0, 0)),
        ],
        out_specs=pl.BlockSpec((BN, D), lambda i: (i, 0)),
        out_shape=jax.ShapeDtypeStruct((N, D), jnp.float32),
    )


_combine_relu = _make_combine("relu")
_combine_tanh = _make_combine("tanh")


def kernel(x, edge_index, g, Wl0, bl0, Wr0, Wl1, bl1, Wr1, Wl2, bl2, Wr2):
    src = edge_index[0]
    dst = edge_index[1]
    zeros_rows = jnp.zeros((RPT, D), jnp.float32)
    zeros_np = jnp.zeros((NP,), jnp.float32)

    p0, cnt2 = _sc_aggregate_count(x, src, dst, zeros_rows, zeros_np)
    cnt = cnt2.reshape(2, NP, 1)

    h = x
    layers = [(Wl0, bl0, Wr0, _combine_relu, None),
              (Wl1, bl1, Wr1, _combine_relu, None),
              (Wl2, bl2, Wr2, _combine_tanh, None)]
    for i, (Wl, bl, Wr, combine, _) in enumerate(layers):
        if i == 0:
            p = p0.reshape(2, NP, D)
        else:
            p = _sc_aggregate(h, src, dst, zeros_rows).reshape(2, NP, D)
        h = combine(p, cnt, h, Wl, bl.reshape(1, D), Wr)
    return h
